# baseline (speedup 1.0000x reference)
import jax
import jax.numpy as jnp
from jax.experimental import pallas as pl
from jax.experimental.pallas import tpu as pltpu

_DIM = 768
_INTER = 256
_N_SHARED = 1
_N_ROUTING = 8
_TOPK = 2
_N_EXPERTS = _N_SHARED + _N_ROUTING
_BLK_T = 1024
_WIDE = _N_EXPERTS * _INTER


def _moe_block_kernel(x_ref, gw_ref, bias_ref, w1_hbm, w2_hbm, w3_hbm, o_ref,
                      w13s, w2s, w1land, w2land, w3land, sems):
    i = pl.program_id(0)

    cp_w1 = pltpu.make_async_copy(w1_hbm, w1land, sems.at[0])
    cp_w2 = pltpu.make_async_copy(w2_hbm, w2land, sems.at[1])
    cp_w3 = pltpu.make_async_copy(w3_hbm, w3land, sems.at[2])

    @pl.when(i == 0)
    def _fetch_and_cast_up_weights():
        cp_w1.start()
        cp_w2.start()
        cp_w3.start()
        cp_w1.wait()
        cp_w3.wait()
        for e in range(_N_EXPERTS):
            base = e * 2 * _INTER
            w13s[:, pl.ds(base, _INTER)] = w1land[e].astype(jnp.bfloat16)
            w13s[:, pl.ds(base + _INTER, _INTER)] = (
                w3land[e].astype(jnp.bfloat16))

    xb = x_ref[...]

    scores = jax.lax.dot_general(
        xb, gw_ref[...], (((1,), (1,)), ((), ())),
        preferred_element_type=jnp.float32)
    scores = scores - jnp.max(scores, axis=-1, keepdims=True)
    es = jnp.exp(scores)
    p = es / jnp.sum(es, axis=-1, keepdims=True)
    sel = p + bias_ref[...]

    lane = jax.lax.broadcasted_iota(jnp.int32, sel.shape, 1)
    big = jnp.int32(_N_ROUTING + 1)

    m1 = jnp.max(sel, axis=-1, keepdims=True)
    i1 = jnp.min(jnp.where(sel >= m1, lane, big), axis=-1, keepdims=True)
    oh1 = (lane == i1).astype(jnp.float32)
    sel2 = sel - oh1 * jnp.float32(1e30)
    m2 = jnp.max(sel2, axis=-1, keepdims=True)
    i2 = jnp.min(jnp.where(sel2 >= m2, lane, big), axis=-1, keepdims=True)
    oh2 = (lane == i2).astype(jnp.float32)

    p1 = jnp.sum(p * oh1, axis=-1, keepdims=True)
    p2 = jnp.sum(p * oh2, axis=-1, keepdims=True)
    wvec16 = (p1 * oh1 + p2 * oh2).astype(jnp.bfloat16)

    xb16 = xb.astype(jnp.bfloat16)
    pieces = []
    for e in range(_N_EXPERTS):
        he = jnp.dot(xb16, w13s[:, e * 2 * _INTER:(e + 1) * 2 * _INTER],
                     preferred_element_type=jnp.float32)
        h1e = he[:, :_INTER]
        h3e = he[:, _INTER:]
        ie = (jax.nn.silu(h1e) * h3e).astype(jnp.bfloat16)
        if e >= _N_SHARED:
            ie = ie * wvec16[:, e - _N_SHARED][:, None]
        pieces.append(ie)
    inter16 = jnp.concatenate(pieces, axis=1)

    @pl.when(i == 0)
    def _cast_down_weights():
        cp_w2.wait()
        for e in range(_N_EXPERTS):
            w2s[pl.ds(e * _INTER, _INTER), :] = w2land[e].astype(jnp.bfloat16)

    o_ref[...] = jnp.dot(inter16, w2s[...], preferred_element_type=jnp.float32)


@jax.jit
def kernel(x, g_w, gate_bias, w1, w2, w3):
    Bb, Tt, C = x.shape
    x2 = x.reshape(Tt, C)
    bias2 = gate_bias.reshape(1, _N_ROUTING)

    grid = (Tt // _BLK_T,)
    out = pl.pallas_call(
        _moe_block_kernel,
        grid=grid,
        in_specs=[
            pl.BlockSpec((_BLK_T, C), lambda i: (i, 0)),
            pl.BlockSpec((_N_ROUTING, C), lambda i: (0, 0)),
            pl.BlockSpec((1, _N_ROUTING), lambda i: (0, 0)),
            pl.BlockSpec(memory_space=pl.ANY),
            pl.BlockSpec(memory_space=pl.ANY),
            pl.BlockSpec(memory_space=pl.ANY),
        ],
        out_specs=pl.BlockSpec((_BLK_T, C), lambda i: (i, 0)),
        out_shape=jax.ShapeDtypeStruct((Tt, C), jnp.float32),
        scratch_shapes=[
            pltpu.VMEM((_DIM, 2 * _WIDE), jnp.bfloat16),
            pltpu.VMEM((_WIDE, _DIM), jnp.bfloat16),
            pltpu.VMEM((_N_EXPERTS, _DIM, _INTER), jnp.float32),
            pltpu.VMEM((_N_EXPERTS, _INTER, _DIM), jnp.float32),
            pltpu.VMEM((_N_EXPERTS, _DIM, _INTER), jnp.float32),
            pltpu.SemaphoreType.DMA((3,)),
        ],
    )(x2, g_w, bias2, w1, w2, w3)
    return out.reshape(Bb, Tt, C)
